# baseline (speedup 1.0000x reference)
import functools

import jax
import jax.numpy as jnp
from jax.experimental import pallas as pl
from jax.experimental.pallas import tpu as pltpu

CHEB_K = 3
TOP_K = 2


def _fused_body(x_ref, emb_w_ref, emb_b_ref, pos_ref, a_ref, gate_w_ref,
                wall_ref, exp_b_ref, o_ref, h_ref, out_ref,
                xt_ref, t2_ref, *, B, T, TH, n_sup, n_exp):
    f32 = jnp.float32
    bf = jnp.bfloat16
    g = pl.program_id(0)
    bidx = g // 2
    half = g % 2

    @pl.when(g < 2 * B)
    def _embed():
        wr = emb_w_ref[...].astype(bf).astype(f32)
        C = wr.shape[0]
        D = wr.shape[1]
        xT = x_ref[0, 0].astype(bf).astype(f32)
        col = pl.ds(bidx * D, D)
        acc = None
        for t in range(TH):
            bias = emb_b_ref[0:1, :] + pos_ref[t]
            val = xT[:, t * C : t * C + 1] * wr[0:1, :] + bias
            for c in range(1, C):
                val = val + xT[:, t * C + c : t * C + c + 1] * wr[c : c + 1, :]
            out_ref[0, t] = val
            acc = val if acc is None else acc + val

        @pl.when(half == 0)
        def _init():
            xt_ref[:, col] = acc

        @pl.when(half == 1)
        def _cont():
            xt_ref[:, col] += acc

    @pl.when(g == 0)
    def _build_t2():
        N = a_ref.shape[1]
        row = jax.lax.broadcasted_iota(jnp.int32, (N, N), 0)
        col = jax.lax.broadcasted_iota(jnp.int32, (N, N), 1)
        eye = (row == col).astype(f32)
        for s in range(n_sup):
            t2 = 2.0 * jnp.dot(a_ref[s], a_ref[s], preferred_element_type=f32) - eye
            t2_ref[s] = t2.astype(bf)

    @pl.when(g == 2 * B)
    def _expert():
        N = xt_ref.shape[0]
        D = emb_w_ref.shape[1]
        xt_bf = (xt_ref[...] / float(T)).astype(bf)

        zs = []
        for s in range(n_sup):
            z1 = jnp.dot(a_ref[s], xt_bf, preferred_element_type=f32)
            z2 = jnp.dot(t2_ref[s], xt_bf, preferred_element_type=f32)
            zs.append((z1.astype(bf), z2.astype(bf)))

        iota = jax.lax.broadcasted_iota(jnp.int32, (N, n_exp), 1)
        for b in range(B):
            sl = slice(b * D, (b + 1) * D)
            xt_b = xt_bf[:, sl]
            chunks = []
            for s in range(n_sup):
                chunks.extend([xt_b, zs[s][0][:, sl], zs[s][1][:, sl]])
            xg = jnp.concatenate(chunks, axis=1)

            gate = jnp.dot(xt_b, gate_w_ref[...], preferred_element_type=f32)
            m1 = jnp.max(gate, axis=1, keepdims=True)
            idx1 = jnp.min(jnp.where(gate == m1, iota, n_exp), axis=1,
                           keepdims=True)
            masked = jnp.where(iota == idx1, -jnp.inf, gate)
            m2 = jnp.max(masked, axis=1, keepdims=True)
            idx2 = jnp.min(jnp.where(masked == m2, iota, n_exp), axis=1,
                           keepdims=True)
            e1 = jnp.exp(m2 - m1)
            denom = 1.0 + e1
            w1 = 1.0 / denom
            w2 = e1 / denom

            oe_all = jnp.dot(xg, wall_ref[...], preferred_element_type=f32)
            o = jnp.zeros((N, D), f32)
            for e in range(n_exp):
                oe = oe_all[:, e * D : (e + 1) * D] + exp_b_ref[e : e + 1, :]
                coef = (jnp.where(idx1 == e, w1, 0.0)
                        + jnp.where(idx2 == e, w2, 0.0))
                o = o + coef * oe
            o_ref[b] = o
            h_ref[b] = jnp.tanh(o)


@jax.jit
def kernel(x, y_cov, supports, emb_W, emb_b, pos_emb, gate_W, exp_W, exp_b):
    B, T, N, C = x.shape
    D = emb_W.shape[1]
    n_sup = supports.shape[0]
    n_exp = exp_W.shape[0]
    bf = jnp.bfloat16

    TH = T // 2
    o_expert, h_expert, current_inputs = pl.pallas_call(
        functools.partial(_fused_body, B=B, T=T, TH=TH, n_sup=n_sup,
                          n_exp=n_exp),
        grid=(2 * B + 1,),
        in_specs=[
            pl.BlockSpec((1, 1, N, TH * C),
                         lambda g: (jnp.minimum(g, 2 * B - 1) // 2,
                                    jnp.minimum(g, 2 * B - 1) % 2, 0, 0)),
            pl.BlockSpec((C, D), lambda g: (0, 0)),
            pl.BlockSpec((1, D), lambda g: (0, 0)),
            pl.BlockSpec((TH, 1, D),
                         lambda g: (jnp.minimum(g, 2 * B - 1) % 2, 0, 0)),
            pl.BlockSpec((n_sup, N, N), lambda g: (0, 0, 0)),
            pl.BlockSpec((D, n_exp), lambda g: (0, 0)),
            pl.BlockSpec((2 * CHEB_K * D, n_exp * D), lambda g: (0, 0)),
            pl.BlockSpec((n_exp, D), lambda g: (0, 0)),
        ],
        out_specs=[
            pl.BlockSpec((B, N, D), lambda g: (0, 0, 0)),
            pl.BlockSpec((B, N, D), lambda g: (0, 0, 0)),
            pl.BlockSpec((1, TH, N, D),
                         lambda g: (jnp.minimum(g, 2 * B - 1) // 2,
                                    jnp.minimum(g, 2 * B - 1) % 2, 0, 0)),
        ],
        out_shape=[
            jax.ShapeDtypeStruct((B, N, D), jnp.float32),
            jax.ShapeDtypeStruct((B, N, D), jnp.float32),
            jax.ShapeDtypeStruct((B, T, N, D), jnp.float32),
        ],
        scratch_shapes=[
            pltpu.VMEM((N, B * D), jnp.float32),
            pltpu.VMEM((n_sup, N, N), jnp.bfloat16),
        ],
    )(x.reshape(B, 2, TH, N, C).transpose(0, 1, 3, 2, 4).reshape(
          B, 2, N, TH * C), emb_W,
      emb_b.reshape(1, D), pos_emb.reshape(T, 1, D), supports.astype(bf),
      gate_W.astype(bf), exp_W.transpose(1, 0, 2).reshape(2 * CHEB_K * D,
                                                          n_exp * D).astype(bf),
      exp_b)

    return (o_expert, h_expert, current_inputs)


# speedup vs baseline: 1.0622x; 1.0622x over previous
import functools

import jax
import jax.numpy as jnp
from jax.experimental import pallas as pl
from jax.experimental.pallas import tpu as pltpu

CHEB_K = 3
TOP_K = 2


def _fused_body(x_ref, emb_w_ref, emb_b_ref, pos_ref, a_ref, gate_w_ref,
                wall_ref, exp_b_ref, o_ref, h_ref, out_ref,
                xt_ref, t2_ref, *, B, T, n_sup, n_exp):
    f32 = jnp.float32
    bf = jnp.bfloat16
    g = pl.program_id(0)

    @pl.when(g < B)
    def _embed():
        wr = emb_w_ref[...].astype(bf).astype(f32)
        C = wr.shape[0]
        D = wr.shape[1]
        xT = x_ref[0].astype(bf).astype(f32)
        acc = None
        for t in range(T):
            bias = emb_b_ref[0:1, :] + pos_ref[t]
            val = xT[:, t * C : t * C + 1] * wr[0:1, :] + bias
            for c in range(1, C):
                val = val + xT[:, t * C + c : t * C + c + 1] * wr[c : c + 1, :]
            out_ref[0, t] = val
            acc = val if acc is None else acc + val
        xt_ref[:, pl.ds(g * D, D)] = acc

    @pl.when(g == 0)
    def _build_t2():
        N = a_ref.shape[1]
        row = jax.lax.broadcasted_iota(jnp.int32, (N, N), 0)
        col = jax.lax.broadcasted_iota(jnp.int32, (N, N), 1)
        eye = (row == col).astype(f32)
        for s in range(n_sup):
            t2 = 2.0 * jnp.dot(a_ref[s], a_ref[s], preferred_element_type=f32) - eye
            t2_ref[s] = t2.astype(bf)

    @pl.when(g == B)
    def _expert():
        N = xt_ref.shape[0]
        D = emb_w_ref.shape[1]
        xt_bf = (xt_ref[...] / float(T)).astype(bf)

        zs = []
        for s in range(n_sup):
            z1 = jnp.dot(a_ref[s], xt_bf, preferred_element_type=f32)
            z2 = jnp.dot(t2_ref[s], xt_bf, preferred_element_type=f32)
            zs.append((z1.astype(bf), z2.astype(bf)))

        iota = jax.lax.broadcasted_iota(jnp.int32, (N, n_exp), 1)
        for b in range(B):
            sl = slice(b * D, (b + 1) * D)
            xt_b = xt_bf[:, sl]
            chunks = []
            for s in range(n_sup):
                chunks.extend([xt_b, zs[s][0][:, sl], zs[s][1][:, sl]])
            xg = jnp.concatenate(chunks, axis=1)

            gate = jnp.dot(xt_b, gate_w_ref[...], preferred_element_type=f32)
            m1 = jnp.max(gate, axis=1, keepdims=True)
            idx1 = jnp.min(jnp.where(gate == m1, iota, n_exp), axis=1,
                           keepdims=True)
            masked = jnp.where(iota == idx1, -jnp.inf, gate)
            m2 = jnp.max(masked, axis=1, keepdims=True)
            idx2 = jnp.min(jnp.where(masked == m2, iota, n_exp), axis=1,
                           keepdims=True)
            e1 = jnp.exp(m2 - m1)
            denom = 1.0 + e1
            w1 = 1.0 / denom
            w2 = e1 / denom

            oe_all = jnp.dot(xg, wall_ref[...], preferred_element_type=f32)
            o = jnp.zeros((N, D), f32)
            for e in range(n_exp):
                oe = oe_all[:, e * D : (e + 1) * D] + exp_b_ref[e : e + 1, :]
                coef = (jnp.where(idx1 == e, w1, 0.0)
                        + jnp.where(idx2 == e, w2, 0.0))
                o = o + coef * oe
            o_ref[b] = o
            h_ref[b] = jnp.tanh(o)


@jax.jit
def kernel(x, y_cov, supports, emb_W, emb_b, pos_emb, gate_W, exp_W, exp_b):
    B, T, N, C = x.shape
    D = emb_W.shape[1]
    n_sup = supports.shape[0]
    n_exp = exp_W.shape[0]
    bf = jnp.bfloat16

    o_expert, h_expert, current_inputs = pl.pallas_call(
        functools.partial(_fused_body, B=B, T=T, n_sup=n_sup, n_exp=n_exp),
        grid=(B + 1,),
        in_specs=[
            pl.BlockSpec((1, N, T * C), lambda g: (jnp.minimum(g, B - 1), 0, 0)),
            pl.BlockSpec((C, D), lambda g: (0, 0)),
            pl.BlockSpec((1, D), lambda g: (0, 0)),
            pl.BlockSpec((T, 1, D), lambda g: (0, 0, 0)),
            pl.BlockSpec((n_sup, N, N), lambda g: (0, 0, 0)),
            pl.BlockSpec((D, n_exp), lambda g: (0, 0)),
            pl.BlockSpec((2 * CHEB_K * D, n_exp * D), lambda g: (0, 0)),
            pl.BlockSpec((n_exp, D), lambda g: (0, 0)),
        ],
        out_specs=[
            pl.BlockSpec((B, N, D), lambda g: (0, 0, 0)),
            pl.BlockSpec((B, N, D), lambda g: (0, 0, 0)),
            pl.BlockSpec((1, T, N, D), lambda g: (jnp.minimum(g, B - 1), 0, 0, 0)),
        ],
        out_shape=[
            jax.ShapeDtypeStruct((B, N, D), jnp.float32),
            jax.ShapeDtypeStruct((B, N, D), jnp.float32),
            jax.ShapeDtypeStruct((B, T, N, D), jnp.float32),
        ],
        scratch_shapes=[
            pltpu.VMEM((N, B * D), jnp.float32),
            pltpu.VMEM((n_sup, N, N), jnp.bfloat16),
        ],
        compiler_params=pltpu.CompilerParams(
            vmem_limit_bytes=110 * 1024 * 1024),
    )(x.transpose(0, 2, 1, 3).reshape(B, N, T * C), emb_W,
      emb_b.reshape(1, D), pos_emb.reshape(T, 1, D), supports.astype(bf),
      gate_W.astype(bf), exp_W.transpose(1, 0, 2).reshape(2 * CHEB_K * D,
                                                          n_exp * D).astype(bf),
      exp_b)

    return (o_expert, h_expert, current_inputs)


# speedup vs baseline: 1.0636x; 1.0014x over previous
import functools

import jax
import jax.numpy as jnp
from jax.experimental import pallas as pl
from jax.experimental.pallas import tpu as pltpu

CHEB_K = 3
TOP_K = 2


def _fused_body(x_ref, emb_w_ref, emb_b_ref, pos_ref, a_ref, gate_w_ref,
                wall_ref, exp_b_ref, o_ref, h_ref, out_ref,
                xt_ref, t2_ref, *, B, T, n_sup, n_exp):
    f32 = jnp.float32
    bf = jnp.bfloat16
    g = pl.program_id(0)

    @pl.when(g < B)
    def _embed():
        wr = emb_w_ref[...].astype(bf).astype(f32)
        C = wr.shape[0]
        D = wr.shape[1]
        xT = x_ref[0].astype(bf).astype(f32)
        acc = None
        for t in range(T):
            bias = emb_b_ref[0:1, :] + pos_ref[t]
            val = xT[:, t * C : t * C + 1] * wr[0:1, :] + bias
            for c in range(1, C):
                val = val + xT[:, t * C + c : t * C + c + 1] * wr[c : c + 1, :]
            out_ref[0, t] = val
            acc = val if acc is None else acc + val
        xt_ref[:, pl.ds(g * D, D)] = acc

    @pl.when(g == 0)
    def _build_t2():
        N = a_ref.shape[1]
        row = jax.lax.broadcasted_iota(jnp.int32, (N, N), 0)
        col = jax.lax.broadcasted_iota(jnp.int32, (N, N), 1)
        eye = (row == col).astype(f32)
        for s in range(n_sup):
            t2 = 2.0 * jnp.dot(a_ref[s], a_ref[s], preferred_element_type=f32) - eye
            t2_ref[s] = t2.astype(bf)

    @pl.when(g == B)
    def _expert():
        N = xt_ref.shape[0]
        D = emb_w_ref.shape[1]
        xt_bf = (xt_ref[...] / float(T)).astype(bf)

        zs = []
        for s in range(n_sup):
            z1 = jnp.dot(a_ref[s], xt_bf, preferred_element_type=f32)
            z2 = jnp.dot(t2_ref[s], xt_bf, preferred_element_type=f32)
            zs.append((z1.astype(bf), z2.astype(bf)))

        iota = jax.lax.broadcasted_iota(jnp.int32, (N, n_exp), 1)
        for b in range(B):
            sl = slice(b * D, (b + 1) * D)
            xt_b = xt_bf[:, sl]
            chunks = []
            for s in range(n_sup):
                chunks.extend([xt_b, zs[s][0][:, sl], zs[s][1][:, sl]])
            xg = jnp.concatenate(chunks, axis=1)

            gate = jnp.dot(xt_b, gate_w_ref[...], preferred_element_type=f32)
            m1 = jnp.max(gate, axis=1, keepdims=True)
            idx1 = jnp.min(jnp.where(gate == m1, iota, n_exp), axis=1,
                           keepdims=True)
            masked = jnp.where(iota == idx1, -jnp.inf, gate)
            m2 = jnp.max(masked, axis=1, keepdims=True)
            idx2 = jnp.min(jnp.where(masked == m2, iota, n_exp), axis=1,
                           keepdims=True)
            e1 = jnp.exp(m2 - m1)
            denom = 1.0 + e1
            w1 = 1.0 / denom
            w2 = e1 / denom

            oe_all = jnp.dot(xg, wall_ref[...], preferred_element_type=f32)
            o = jnp.zeros((N, D), f32)
            for e in range(n_exp):
                oe = oe_all[:, e * D : (e + 1) * D] + exp_b_ref[e : e + 1, :]
                coef = (jnp.where(idx1 == e, w1, 0.0)
                        + jnp.where(idx2 == e, w2, 0.0))
                o = o + coef * oe
            o_ref[b] = o
            h_ref[b] = jnp.tanh(o)


@jax.jit
def kernel(x, y_cov, supports, emb_W, emb_b, pos_emb, gate_W, exp_W, exp_b):
    B, T, N, C = x.shape
    D = emb_W.shape[1]
    n_sup = supports.shape[0]
    n_exp = exp_W.shape[0]
    bf = jnp.bfloat16

    o_expert, h_expert, current_inputs = pl.pallas_call(
        functools.partial(_fused_body, B=B, T=T, n_sup=n_sup, n_exp=n_exp),
        grid=(B + 1,),
        in_specs=[
            pl.BlockSpec((1, N, T * C), lambda g: (jnp.minimum(g, B - 1), 0, 0)),
            pl.BlockSpec((C, D), lambda g: (0, 0)),
            pl.BlockSpec((1, D), lambda g: (0, 0)),
            pl.BlockSpec((T, 1, D), lambda g: (0, 0, 0)),
            pl.BlockSpec((n_sup, N, N), lambda g: (0, 0, 0)),
            pl.BlockSpec((D, n_exp), lambda g: (0, 0)),
            pl.BlockSpec((2 * CHEB_K * D, n_exp * D), lambda g: (0, 0)),
            pl.BlockSpec((n_exp, D), lambda g: (0, 0)),
        ],
        out_specs=[
            pl.BlockSpec((B, N, D), lambda g: (0, 0, 0)),
            pl.BlockSpec((B, N, D), lambda g: (0, 0, 0)),
            pl.BlockSpec((1, T, N, D), lambda g: (jnp.minimum(g, B - 1), 0, 0, 0)),
        ],
        out_shape=[
            jax.ShapeDtypeStruct((B, N, D), jnp.float32),
            jax.ShapeDtypeStruct((B, N, D), jnp.float32),
            jax.ShapeDtypeStruct((B, T, N, D), jnp.float32),
        ],
        scratch_shapes=[
            pltpu.VMEM((N, B * D), jnp.float32),
            pltpu.VMEM((n_sup, N, N), jnp.bfloat16),
        ],
    )(x.transpose(0, 2, 1, 3).reshape(B, N, T * C), emb_W,
      emb_b.reshape(1, D), pos_emb.reshape(T, 1, D), supports.astype(bf),
      gate_W.astype(bf), exp_W.transpose(1, 0, 2).reshape(2 * CHEB_K * D,
                                                          n_exp * D).astype(bf),
      exp_b)

    return (o_expert, h_expert, current_inputs)
